# baseline (speedup 1.0000x reference)
import functools

import jax
import jax.numpy as jnp
from jax.experimental import pallas as pl
from jax.experimental.pallas import tpu as pltpu

_Q = 32
_D = 128
_MD = 768
_K = 64
_C = 8192
_NB = 128
_T1 = 2
_PW = _T1 * _NB
_DEPTH = 6

_NEG = float("-inf")
_IMAX = 2**31 - 1


def _chunk_start(n_docs, j):
    return jnp.minimum(j * _C, n_docs - _C)


def _body(n_docs, n_chunks, fin_w, q_ref, w_ref, b_ref, docs_hbm, out_ref,
          ring, sems, s_ref, qe_ref, topv_ref, topi_ref,
          p_ref, pi_ref, fv_ref, fg_ref, kf_ref):
    step = pl.program_id(0)

    def copy(j, slot):
        return pltpu.make_async_copy(
            docs_hbm.at[pl.ds(_chunk_start(n_docs, j), _C), :],
            ring.at[slot], sems.at[slot])

    @pl.when(step == 0)
    def _init():
        qe = jax.lax.dot_general(
            q_ref[...], w_ref[...],
            dimension_numbers=(((1,), (1,)), ((), ())),
            preferred_element_type=jnp.float32)
        qe_ref[...] = qe + b_ref[...]
        for j in range(_DEPTH):
            if j < n_chunks:
                copy(j, j).start()

    slot = jax.lax.rem(step, _DEPTH)
    copy(step, slot).wait()

    start = _chunk_start(n_docs, step)
    colc = jax.lax.broadcasted_iota(jnp.int32, (_Q, _C), 1)

    scores = jax.lax.dot_general(
        qe_ref[...], ring[slot],
        dimension_numbers=(((1,), (1,)), ((), ())),
        preferred_element_type=jnp.float32)
    s3 = jnp.where(colc + start >= step * _C, scores, _NEG
                   ).reshape(_Q, _C // _NB, _NB)

    @pl.when(step + _DEPTH < n_chunks)
    def _prefetch():
        copy(step + _DEPTH, slot).start()

    nm = _C // _NB
    im = jax.lax.broadcasted_iota(jnp.int32, (_Q, nm, _NB), 1)
    biota = jax.lax.broadcasted_iota(jnp.int32, (_Q, _NB), 1)
    ms, gs = [], []
    for _lvl in range(_T1):
        m = jnp.max(s3, axis=1)
        il = jnp.min(jnp.where(s3 == m[:, None, :], im, nm), axis=1)
        s3 = jnp.where(im == il[:, None, :], _NEG, s3)
        ms.append(m)
        gs.append(start + il * _NB + biota)
    p_ref[step] = jnp.concatenate(ms, axis=1)
    pi_ref[step] = jnp.concatenate(gs, axis=1)

    @pl.when(step == n_chunks - 1)
    def _finish():
        lane = jax.lax.broadcasted_iota(jnp.int32, (_Q, _K), 1)
        lfin = jax.lax.broadcasted_iota(jnp.int32, (_Q, fin_w), 1)

        fv_ref[...] = jnp.full((_Q, fin_w), _NEG, jnp.float32)
        fg_ref[...] = jnp.zeros((_Q, fin_w), jnp.int32)
        for c in range(n_chunks):
            fv_ref[:, _PW * c:_PW * (c + 1)] = p_ref[c]
            fg_ref[:, _PW * c:_PW * (c + 1)] = pi_ref[c]
        kf_ref[...] = fv_ref[...]

        def pop(i, carry):
            fv = fv_ref[...]
            fg = fg_ref[...]
            v = jnp.max(fv, axis=1, keepdims=True)
            sel = fv == v
            g = jnp.min(jnp.where(sel, fg, _IMAX), axis=1, keepdims=True)
            fv_ref[...] = jnp.where(sel & (fg == g), _NEG, fv)
            topi_ref[...] = jnp.where(lane == i, g, topi_ref[...])
            return carry

        jax.lax.fori_loop(0, _K, pop, 0)

        deep = (lfin % _PW) >= (_T1 - 1) * _NB
        risky = jnp.any((fv_ref[...] == _NEG) & (kf_ref[...] > _NEG) & deep)

        @pl.when(risky)
        def _fallback():
            topv_ref[...] = jnp.full((_Q, _K), _NEG, jnp.float32)
            topi_ref[...] = jnp.zeros((_Q, _K), jnp.int32)

            def do_chunk(c, carry):
                cst = _chunk_start(n_docs, c)
                copy(c, 0).start()
                copy(c, 0).wait()
                sc = jax.lax.dot_general(
                    qe_ref[...], ring[0],
                    dimension_numbers=(((1,), (1,)), ((), ())),
                    preferred_element_type=jnp.float32)
                s_ref[...] = jnp.where(colc + cst >= c * _C, sc, _NEG)

                vmax0 = jnp.max(s_ref[...], axis=1, keepdims=True)
                tau0 = topv_ref[:, _K - 1:_K]
                cnt = jnp.minimum(
                    jnp.max(jnp.sum((s_ref[...] > tau0).astype(jnp.int32),
                                    axis=1)), _K)

                def ins(_, vmax):
                    s = s_ref[...]
                    imax = jnp.min(jnp.where(s == vmax, colc, _C), axis=1,
                                   keepdims=True)
                    s = jnp.where(colc == imax, _NEG, s)
                    s_ref[...] = s
                    gidx = (imax + cst).astype(jnp.int32)
                    topv = topv_ref[...]
                    topi = topi_ref[...]
                    pos = jnp.sum((topv >= vmax).astype(jnp.int32), axis=1,
                                  keepdims=True)
                    sv = jnp.concatenate([topv[:, :1], topv[:, :_K - 1]],
                                         axis=1)
                    si = jnp.concatenate([topi[:, :1], topi[:, :_K - 1]],
                                         axis=1)
                    topv_ref[...] = jnp.where(
                        lane < pos, topv, jnp.where(lane == pos, vmax, sv))
                    topi_ref[...] = jnp.where(
                        lane < pos, topi, jnp.where(lane == pos, gidx, si))
                    return jnp.max(s, axis=1, keepdims=True)

                jax.lax.fori_loop(0, cnt, ins, vmax0)
                return carry

            jax.lax.fori_loop(0, n_chunks, do_chunk, 0)

        out_ref[...] = topi_ref[...]


def kernel(query, W, b, doc_records, top_k):
    n_docs = doc_records.shape[0]
    n_chunks = pl.cdiv(n_docs, _C)
    fin_w = -(-(n_chunks * _PW) // 128) * 128
    b2d = b.reshape(1, _D)

    out = pl.pallas_call(
        functools.partial(_body, n_docs, n_chunks, fin_w),
        grid=(n_chunks,),
        in_specs=[
            pl.BlockSpec((_Q, _MD), lambda i: (0, 0)),
            pl.BlockSpec((_D, _MD), lambda i: (0, 0)),
            pl.BlockSpec((1, _D), lambda i: (0, 0)),
            pl.BlockSpec(memory_space=pl.ANY),
        ],
        out_specs=pl.BlockSpec((_Q, _K), lambda i: (0, 0)),
        out_shape=jax.ShapeDtypeStruct((_Q, _K), jnp.int32),
        scratch_shapes=[
            pltpu.VMEM((_DEPTH, _C, _D), jnp.float32),
            pltpu.SemaphoreType.DMA((_DEPTH,)),
            pltpu.VMEM((_Q, _C), jnp.float32),
            pltpu.VMEM((_Q, _D), jnp.float32),
            pltpu.VMEM((_Q, _K), jnp.float32),
            pltpu.VMEM((_Q, _K), jnp.int32),
            pltpu.VMEM((n_chunks, _Q, _PW), jnp.float32),
            pltpu.VMEM((n_chunks, _Q, _PW), jnp.int32),
            pltpu.VMEM((_Q, fin_w), jnp.float32),
            pltpu.VMEM((_Q, fin_w), jnp.int32),
            pltpu.VMEM((_Q, fin_w), jnp.float32),
        ],
        compiler_params=pltpu.CompilerParams(
            dimension_semantics=("arbitrary",)),
    )(query, W, b2d, doc_records)
    return out + (top_k - top_k)


# speedup vs baseline: 2.7645x; 2.7645x over previous
import functools

import jax
import jax.numpy as jnp
from jax.experimental import pallas as pl
from jax.experimental.pallas import tpu as pltpu

_Q = 32
_D = 128
_MD = 768
_K = 64
_C = 8192
_NB = 128
_T1 = 3
_PW = _T1 * _NB
_DEPTH = 4

_NEG = float("-inf")
_IMAX = 2**31 - 1


def _chunk_start(n_docs, j):
    return jnp.minimum(j * _C, n_docs - _C)


def _body(n_docs, n_chunks, fin_w, q_ref, w_ref, b_ref, docs_hbm, out_ref,
          ring, sems, s_ref, qe_ref, topv_ref, topi_ref,
          p_ref, pi_ref, fv_ref, fg_ref, kf_ref):
    step = pl.program_id(0)

    def copy(j, slot):
        return pltpu.make_async_copy(
            docs_hbm.at[pl.ds(_chunk_start(n_docs, j), _C), :],
            ring.at[slot], sems.at[slot])

    @pl.when(step == 0)
    def _init():
        qe = jax.lax.dot_general(
            q_ref[...], w_ref[...],
            dimension_numbers=(((1,), (1,)), ((), ())),
            preferred_element_type=jnp.float32)
        qe_ref[...] = qe + b_ref[...]
        for j in range(_DEPTH):
            if j < n_chunks:
                copy(j, j).start()

    slot = jax.lax.rem(step, _DEPTH)
    copy(step, slot).wait()

    start = _chunk_start(n_docs, step)
    colc = jax.lax.broadcasted_iota(jnp.int32, (_Q, _C), 1)

    scores = jax.lax.dot_general(
        qe_ref[...], ring[slot],
        dimension_numbers=(((1,), (1,)), ((), ())),
        preferred_element_type=jnp.float32)
    s3 = jnp.where(colc + start >= step * _C, scores, _NEG
                   ).reshape(_Q, _C // _NB, _NB)

    @pl.when(step + _DEPTH < n_chunks)
    def _prefetch():
        copy(step + _DEPTH, slot).start()

    nm = _C // _NB
    im = jax.lax.broadcasted_iota(jnp.int32, (_Q, nm, _NB), 1)
    biota = jax.lax.broadcasted_iota(jnp.int32, (_Q, _NB), 1)
    ms, gs = [], []
    for _lvl in range(_T1):
        m = jnp.max(s3, axis=1)
        il = jnp.min(jnp.where(s3 == m[:, None, :], im, nm), axis=1)
        s3 = jnp.where(im == il[:, None, :], _NEG, s3)
        ms.append(m)
        gs.append(start + il * _NB + biota)
    p_ref[step] = jnp.concatenate(ms, axis=1)
    pi_ref[step] = jnp.concatenate(gs, axis=1)

    @pl.when(step == n_chunks - 1)
    def _finish():
        lane = jax.lax.broadcasted_iota(jnp.int32, (_Q, _K), 1)
        lfin = jax.lax.broadcasted_iota(jnp.int32, (_Q, fin_w), 1)

        fv_ref[...] = jnp.full((_Q, fin_w), _NEG, jnp.float32)
        fg_ref[...] = jnp.zeros((_Q, fin_w), jnp.int32)
        for c in range(n_chunks):
            fv_ref[:, _PW * c:_PW * (c + 1)] = p_ref[c]
            fg_ref[:, _PW * c:_PW * (c + 1)] = pi_ref[c]
        kf_ref[...] = fv_ref[...]

        def pop(i, carry):
            fv = fv_ref[...]
            fg = fg_ref[...]
            v = jnp.max(fv, axis=1, keepdims=True)
            sel = fv == v
            g = jnp.min(jnp.where(sel, fg, _IMAX), axis=1, keepdims=True)
            fv_ref[...] = jnp.where(sel & (fg == g), _NEG, fv)
            topi_ref[...] = jnp.where(lane == i, g, topi_ref[...])
            return carry

        jax.lax.fori_loop(0, _K, pop, 0)

        deep = (lfin % _PW) >= (_T1 - 1) * _NB
        risky = jnp.any((fv_ref[...] == _NEG) & (kf_ref[...] > _NEG) & deep)

        @pl.when(risky)
        def _fallback():
            topv_ref[...] = jnp.full((_Q, _K), _NEG, jnp.float32)
            topi_ref[...] = jnp.zeros((_Q, _K), jnp.int32)

            def do_chunk(c, carry):
                cst = _chunk_start(n_docs, c)
                copy(c, 0).start()
                copy(c, 0).wait()
                sc = jax.lax.dot_general(
                    qe_ref[...], ring[0],
                    dimension_numbers=(((1,), (1,)), ((), ())),
                    preferred_element_type=jnp.float32)
                s_ref[...] = jnp.where(colc + cst >= c * _C, sc, _NEG)

                vmax0 = jnp.max(s_ref[...], axis=1, keepdims=True)
                tau0 = topv_ref[:, _K - 1:_K]
                cnt = jnp.minimum(
                    jnp.max(jnp.sum((s_ref[...] > tau0).astype(jnp.int32),
                                    axis=1)), _K)

                def ins(_, vmax):
                    s = s_ref[...]
                    imax = jnp.min(jnp.where(s == vmax, colc, _C), axis=1,
                                   keepdims=True)
                    s = jnp.where(colc == imax, _NEG, s)
                    s_ref[...] = s
                    gidx = (imax + cst).astype(jnp.int32)
                    topv = topv_ref[...]
                    topi = topi_ref[...]
                    pos = jnp.sum((topv >= vmax).astype(jnp.int32), axis=1,
                                  keepdims=True)
                    sv = jnp.concatenate([topv[:, :1], topv[:, :_K - 1]],
                                         axis=1)
                    si = jnp.concatenate([topi[:, :1], topi[:, :_K - 1]],
                                         axis=1)
                    topv_ref[...] = jnp.where(
                        lane < pos, topv, jnp.where(lane == pos, vmax, sv))
                    topi_ref[...] = jnp.where(
                        lane < pos, topi, jnp.where(lane == pos, gidx, si))
                    return jnp.max(s, axis=1, keepdims=True)

                jax.lax.fori_loop(0, cnt, ins, vmax0)
                return carry

            jax.lax.fori_loop(0, n_chunks, do_chunk, 0)

        out_ref[...] = topi_ref[...]


def kernel(query, W, b, doc_records, top_k):
    n_docs = doc_records.shape[0]
    n_chunks = pl.cdiv(n_docs, _C)
    fin_w = -(-(n_chunks * _PW) // 128) * 128
    b2d = b.reshape(1, _D)

    out = pl.pallas_call(
        functools.partial(_body, n_docs, n_chunks, fin_w),
        grid=(n_chunks,),
        in_specs=[
            pl.BlockSpec((_Q, _MD), lambda i: (0, 0)),
            pl.BlockSpec((_D, _MD), lambda i: (0, 0)),
            pl.BlockSpec((1, _D), lambda i: (0, 0)),
            pl.BlockSpec(memory_space=pl.ANY),
        ],
        out_specs=pl.BlockSpec((_Q, _K), lambda i: (0, 0)),
        out_shape=jax.ShapeDtypeStruct((_Q, _K), jnp.int32),
        scratch_shapes=[
            pltpu.VMEM((_DEPTH, _C, _D), jnp.float32),
            pltpu.SemaphoreType.DMA((_DEPTH,)),
            pltpu.VMEM((_Q, _C), jnp.float32),
            pltpu.VMEM((_Q, _D), jnp.float32),
            pltpu.VMEM((_Q, _K), jnp.float32),
            pltpu.VMEM((_Q, _K), jnp.int32),
            pltpu.VMEM((n_chunks, _Q, _PW), jnp.float32),
            pltpu.VMEM((n_chunks, _Q, _PW), jnp.int32),
            pltpu.VMEM((_Q, fin_w), jnp.float32),
            pltpu.VMEM((_Q, fin_w), jnp.int32),
            pltpu.VMEM((_Q, fin_w), jnp.float32),
        ],
        compiler_params=pltpu.CompilerParams(
            dimension_semantics=("arbitrary",)),
    )(query, W, b2d, doc_records)
    return out + (top_k - top_k)
